# baseline (speedup 1.0000x reference)
import functools

import jax
import jax.numpy as jnp
from jax import lax
from jax.experimental import pallas as pl
from jax.experimental.pallas import tpu as pltpu
from jax.experimental.pallas import tpu_sc as plsc

NC = 2
NS = 16
LANES = 128


def _sc_segment_sum(table, eidx_f, eidx_s, na, cf, cs, grp):
  n, d = table.shape
  nchf, nchs = eidx_f.shape[1], eidx_s.shape[1]
  ngf = nchf // grp
  assert nchf == ngf * grp and ngf % 2 == 0
  rpt = na // NS
  mesh = plsc.VectorSubcoreMesh(core_axis_name="c", subcore_axis_name="s")

  cmax = max(cf, cs)

  @functools.partial(
      pl.kernel,
      out_type=jax.ShapeDtypeStruct((NC, na, d), jnp.float32),
      mesh=mesh,
      compiler_params=pltpu.CompilerParams(use_tc_tiling_on_sc=False),
      scratch_types=[
          pltpu.VMEM((2, grp, 2, cf), jnp.int32),
          pltpu.VMEM((nchs, 2, cs), jnp.int32),
          pltpu.VMEM((cmax, d), jnp.float32),
          pltpu.VMEM_SHARED((na, d), jnp.float32),
          [pltpu.SemaphoreType.DMA] * 2,
          pltpu.SemaphoreType.DMA,
      ],
  )
  def body(table_hbm, eidxf_hbm, eidxs_hbm, out_hbm, idxf_v, idxs_v,
           rows_v, acc_sh, isems, gsem):
    cid = lax.axis_index("c")
    sid = lax.axis_index("s")

    dchunks = d // 16

    def zb(i, _):
      r = i // dchunks
      col = (i % dchunks) * 16
      rows_v[r, pl.ds(col, 16)] = jnp.zeros((16,), jnp.float32)
      return 0

    lax.fori_loop(0, LANES * dchunks, zb, 0)
    base = sid * rpt
    for k in range(rpt // LANES):
      pltpu.sync_copy(rows_v.at[pl.ds(0, LANES)],
                      acc_sh.at[pl.ds(base + k * LANES, LANES)])
    zrem = rpt % LANES
    if zrem:
      pltpu.sync_copy(
          rows_v.at[pl.ds(0, zrem)],
          acc_sh.at[pl.ds(base + (rpt // LANES) * LANES, zrem)])

    def run_slow(eidx_hbm, idx_v, nch, c):
      pltpu.sync_copy(eidx_hbm.at[sid], idx_v)
      rv = rows_v.at[pl.ds(0, c)]

      def jb(j, _):
        pltpu.async_copy(table_hbm.at[idx_v.at[j, 0]], rv, gsem)
        pltpu.make_async_copy(table_hbm.at[idx_v.at[j, 0]], rv,
                              gsem).wait()
        pltpu.sync_copy(rv, acc_sh.at[idx_v.at[j, 1]], add=True)
        return 0

      lax.fori_loop(0, nch, jb, 0)

    def run_core(eidx_hbm, idx_v, ngrp, c):
      pltpu.async_copy(eidx_hbm.at[sid, pl.ds(0, grp)], idx_v.at[0],
                       isems[0])
      pltpu.async_copy(eidx_hbm.at[sid, pl.ds(grp, grp)], idx_v.at[1],
                       isems[1])
      rv = rows_v.at[pl.ds(0, c)]

      def gb(gg, _):
        for p in range(2):
          g = 2 * gg + p
          pltpu.make_async_copy(eidx_hbm.at[sid, pl.ds(0, grp)],
                                idx_v.at[p], isems[p]).wait()
          for k in range(grp):
            pltpu.async_copy(table_hbm.at[idx_v.at[p, k, 0]], rv, gsem)
            pltpu.make_async_copy(table_hbm.at[idx_v.at[p, k, 0]], rv,
                                  gsem).wait()
            pltpu.sync_copy(rv, acc_sh.at[idx_v.at[p, k, 1]], add=True)

          @pl.when(g + 2 < ngrp)
          def _(p=p, g=g):
            pltpu.async_copy(eidx_hbm.at[sid, pl.ds((g + 2) * grp, grp)],
                             idx_v.at[p], isems[p])

        return 0

      lax.fori_loop(0, ngrp // 2, gb, 0)

    @pl.when(cid == 0)
    def _():
      run_core(eidxf_hbm, idxf_v, ngf, cf)

    @pl.when(cid == 1)
    def _():
      run_slow(eidxs_hbm, idxs_v, nchs, cs)

    plsc.subcore_barrier()
    pltpu.sync_copy(acc_sh.at[pl.ds(base, rpt)],
                    out_hbm.at[cid].at[pl.ds(base, rpt)])

  return body(table, eidx_f, eidx_s)


def _row_spec(r, width):
  return pl.BlockSpec((r, width), lambda i: (i, 0))


def _full_spec(shape):
  return pl.BlockSpec(shape, lambda i: tuple(0 for _ in shape))


def _tc1(parts, x, w1_rel, b1, w1_root, w2_rel, r):
  n, din = x.shape
  h1 = w1_rel.shape[1]
  h2w = w2_rel.shape[1]
  d = parts.shape[2]

  def body(p_ref, x_ref, w1r, b1r, w1o, w2r, h_ref, t_ref, rdeg_ref):
    agg = p_ref[0] + p_ref[1]
    deg = agg[:, din:din + 1]
    rdeg = 1.0 / jnp.maximum(deg, 1.0)
    mean1 = agg[:, :din] * rdeg
    h = jnp.maximum(
        jnp.dot(mean1, w1r[...], preferred_element_type=jnp.float32)
        + b1r[...]
        + jnp.dot(x_ref[...], w1o[...], preferred_element_type=jnp.float32),
        0.0)
    h_ref[...] = h
    t_ref[...] = jnp.dot(h, w2r[...], preferred_element_type=jnp.float32)
    rdeg_ref[...] = rdeg

  return pl.pallas_call(
      body,
      grid=(n // r,),
      in_specs=[
          pl.BlockSpec((2, r, d), lambda i: (0, i, 0)),
          _row_spec(r, din),
          _full_spec(w1_rel.shape),
          _full_spec(b1.shape),
          _full_spec(w1_root.shape),
          _full_spec(w2_rel.shape),
      ],
      out_specs=[_row_spec(r, h1), _row_spec(r, h2w), _row_spec(r, 1)],
      out_shape=[
          jax.ShapeDtypeStruct((n, h1), jnp.float32),
          jax.ShapeDtypeStruct((n, h2w), jnp.float32),
          jax.ShapeDtypeStruct((n, 1), jnp.float32),
      ],
  )(parts, x, w1_rel, b1, w1_root, w2_rel)


def _tc2(parts, h, rdeg, b2, w2_root, wmuls, r):
  n, h1 = h.shape
  d = parts.shape[2]
  oc2 = wmuls.shape[1]

  def body(p_ref, h_ref, rdeg_ref, b2r, w2o, wm, h2_ref, pout_ref):
    mean2 = (p_ref[0] + p_ref[1]) * rdeg_ref[...]
    hh2 = jnp.maximum(
        mean2 + b2r[...]
        + jnp.dot(h_ref[...], w2o[...], preferred_element_type=jnp.float32),
        0.0)
    h2_ref[...] = hh2
    pout_ref[...] = jnp.dot(hh2, wm[...], preferred_element_type=jnp.float32)

  return pl.pallas_call(
      body,
      grid=(n // r,),
      in_specs=[
          pl.BlockSpec((2, r, d), lambda i: (0, i, 0)),
          _row_spec(r, h1),
          _row_spec(r, 1),
          _full_spec(b2.shape),
          _full_spec(w2_root.shape),
          _full_spec(wmuls.shape),
      ],
      out_specs=[_row_spec(r, d), _row_spec(r, oc2)],
      out_shape=[
          jax.ShapeDtypeStruct((n, d), jnp.float32),
          jax.ShapeDtypeStruct((n, oc2), jnp.float32),
      ],
  )(parts, h, rdeg, b2, w2_root, wmuls)


def _tc3(parts, h2, rdeg, bmuls, wroots, r):
  n, hd = h2.shape
  oc2 = parts.shape[2]

  def body(p_ref, h2_ref, rdeg_ref, br, wr, out_ref):
    mean3 = (p_ref[0] + p_ref[1]) * rdeg_ref[...]
    out_ref[...] = (
        mean3 + br[...]
        + jnp.dot(h2_ref[...], wr[...], preferred_element_type=jnp.float32))

  return pl.pallas_call(
      body,
      grid=(n // r,),
      in_specs=[
          pl.BlockSpec((2, r, oc2), lambda i: (0, i, 0)),
          _row_spec(r, hd),
          _row_spec(r, 1),
          _full_spec(bmuls.shape),
          _full_spec(wroots.shape),
      ],
      out_specs=_row_spec(r, oc2),
      out_shape=jax.ShapeDtypeStruct((n, oc2), jnp.float32),
  )(parts, h2, rdeg, bmuls, wroots)


def kernel(x, edge_index, w1_rel, b1, w1_root, w2_rel, b2, w2_root,
           wmu_rel, bmu, wmu_root, wls_rel, bls, wls_root):
  n, din = x.shape
  e = edge_index.shape[1]
  grp = 2
  c12f, c12s = 128, 128
  c3f, c3s = 512, 512
  q = 2048
  ep = ((e + NS * q - 1) // (NS * q)) * (NS * q)
  per_tile = ep // NS
  n1 = max(q, round(0.3 * per_tile / q) * q)
  n0 = per_tile - n1
  assert n0 % 2048 == 0 and n1 % 2048 == 0
  na = ((n + 1 + NS - 1) // NS) * NS

  src = edge_index[0]
  dst = edge_index[1]
  pad = ep - e
  if pad:
    src = jnp.concatenate([src, jnp.zeros((pad,), src.dtype)])
    dst = jnp.concatenate([dst, jnp.full((pad,), n, dst.dtype)])

  sf, ss = src[:NS * n0], src[NS * n0:]
  df, ds_ = dst[:NS * n0], dst[NS * n0:]

  def mk_eidx(s_, d_, c):
    s4 = s_.reshape(NS, -1, c)
    d4 = d_.reshape(NS, -1, c)
    return jnp.stack([s4, d4], axis=2)

  eidx12f = mk_eidx(sf, df, c12f)
  eidx12s = mk_eidx(ss, ds_, c12s)
  eidx3f = mk_eidx(sf, df, c3f)
  eidx3s = mk_eidx(ss, ds_, c3s)

  r = 2000 if n % 2000 == 0 else 8 * (n // 8)
  t1 = jnp.concatenate(
      [x, jnp.ones((n, 1), x.dtype), jnp.zeros((n, 15), x.dtype)], axis=1)
  p1 = _sc_segment_sum(t1, eidx12f, eidx12s, na, c12f, c12s, grp)
  h, t, rdeg = _tc1(p1, x, w1_rel, b1.reshape(1, -1), w1_root, w2_rel, r)

  p2 = _sc_segment_sum(t, eidx12f, eidx12s, na, c12f, c12s, grp)
  wmuls = jnp.concatenate([wmu_rel, wls_rel], axis=1)
  h2, p = _tc2(p2, h, rdeg, b2.reshape(1, -1), w2_root, wmuls, r)

  p3 = _sc_segment_sum(p, eidx3f, eidx3s, na, c3f, c3s, grp)
  wroots = jnp.concatenate([wmu_root, wls_root], axis=1)
  bmuls = jnp.concatenate([bmu, bls]).reshape(1, -1)
  out = _tc3(p3, h2, rdeg, bmuls, wroots, r)
  oc = wmu_rel.shape[1]
  return out[:, :oc], out[:, oc:]


# speedup vs baseline: 1.2086x; 1.2086x over previous
import functools

import jax
import jax.numpy as jnp
from jax import lax
from jax.experimental import pallas as pl
from jax.experimental.pallas import tpu as pltpu
from jax.experimental.pallas import tpu_sc as plsc

NC = 2
NS = 16
LANES = 128


def _sc_segment_sum(table, src3, dst3, na):
  n, d = table.shape
  nw, nch, _ = src3.shape
  rpt = na // NS
  mesh = plsc.VectorSubcoreMesh(core_axis_name="c", subcore_axis_name="s")

  @functools.partial(
      pl.kernel,
      out_type=jax.ShapeDtypeStruct((NC, na, d), jnp.float32),
      mesh=mesh,
      compiler_params=pltpu.CompilerParams(use_tc_tiling_on_sc=False),
      scratch_types=[
          pltpu.VMEM((nch, LANES), jnp.int32),
          pltpu.VMEM((nch, LANES), jnp.int32),
          pltpu.VMEM((LANES, d), jnp.float32),
          pltpu.VMEM_SHARED((na, d), jnp.float32),
          pltpu.SemaphoreType.DMA,
      ],
  )
  def body(table_hbm, src_hbm, dst_hbm, out_hbm, src_v, dst_v, rows_v,
           acc_sh, sem):
    cid = lax.axis_index("c")
    sid = lax.axis_index("s")
    wid = cid * NS + sid

    dchunks = d // 16

    def zb(i, _):
      r = i // dchunks
      col = (i % dchunks) * 16
      rows_v[r, pl.ds(col, 16)] = jnp.zeros((16,), jnp.float32)
      return 0

    lax.fori_loop(0, LANES * dchunks, zb, 0)
    base = sid * rpt
    for k in range(rpt // LANES):
      pltpu.sync_copy(rows_v, acc_sh.at[pl.ds(base + k * LANES, LANES)])

    pltpu.sync_copy(src_hbm.at[wid], src_v)
    pltpu.sync_copy(dst_hbm.at[wid], dst_v)
    plsc.subcore_barrier()

    def eb(j, _):
      pltpu.async_copy(table_hbm.at[src_v.at[j]], rows_v, sem).wait()
      pltpu.sync_copy(rows_v, acc_sh.at[dst_v.at[j]], add=True)
      return 0

    lax.fori_loop(0, nch, eb, 0)
    plsc.subcore_barrier()
    pltpu.sync_copy(acc_sh.at[pl.ds(base, rpt)],
                    out_hbm.at[cid].at[pl.ds(base, rpt)])

  return body(table, src3, dst3)


def _row_spec(r, width):
  return pl.BlockSpec((r, width), lambda i: (i, 0))


def _full_spec(shape):
  return pl.BlockSpec(shape, lambda i: tuple(0 for _ in shape))


def _tc1(parts, x, w1_rel, b1, w1_root, w2_rel, r):
  n, din = x.shape
  h1 = w1_rel.shape[1]
  h2w = w2_rel.shape[1]
  d = parts.shape[2]

  def body(p_ref, x_ref, w1r, b1r, w1o, w2r, h_ref, t_ref, rdeg_ref):
    agg = p_ref[0] + p_ref[1]
    deg = agg[:, din:din + 1]
    rdeg = 1.0 / jnp.maximum(deg, 1.0)
    mean1 = agg[:, :din] * rdeg
    h = jnp.maximum(
        jnp.dot(mean1, w1r[...], preferred_element_type=jnp.float32)
        + b1r[...]
        + jnp.dot(x_ref[...], w1o[...], preferred_element_type=jnp.float32),
        0.0)
    h_ref[...] = h
    t_ref[...] = jnp.dot(h, w2r[...], preferred_element_type=jnp.float32)
    rdeg_ref[...] = rdeg

  return pl.pallas_call(
      body,
      grid=(n // r,),
      in_specs=[
          pl.BlockSpec((2, r, d), lambda i: (0, i, 0)),
          _row_spec(r, din),
          _full_spec(w1_rel.shape),
          _full_spec(b1.shape),
          _full_spec(w1_root.shape),
          _full_spec(w2_rel.shape),
      ],
      out_specs=[_row_spec(r, h1), _row_spec(r, h2w), _row_spec(r, 1)],
      out_shape=[
          jax.ShapeDtypeStruct((n, h1), jnp.float32),
          jax.ShapeDtypeStruct((n, h2w), jnp.float32),
          jax.ShapeDtypeStruct((n, 1), jnp.float32),
      ],
  )(parts, x, w1_rel, b1, w1_root, w2_rel)


def _tc2(parts, h, rdeg, b2, w2_root, wmuls, r):
  n, h1 = h.shape
  d = parts.shape[2]
  oc2 = wmuls.shape[1]

  def body(p_ref, h_ref, rdeg_ref, b2r, w2o, wm, h2_ref, pout_ref):
    mean2 = (p_ref[0] + p_ref[1]) * rdeg_ref[...]
    hh2 = jnp.maximum(
        mean2 + b2r[...]
        + jnp.dot(h_ref[...], w2o[...], preferred_element_type=jnp.float32),
        0.0)
    h2_ref[...] = hh2
    pout_ref[...] = jnp.dot(hh2, wm[...], preferred_element_type=jnp.float32)

  return pl.pallas_call(
      body,
      grid=(n // r,),
      in_specs=[
          pl.BlockSpec((2, r, d), lambda i: (0, i, 0)),
          _row_spec(r, h1),
          _row_spec(r, 1),
          _full_spec(b2.shape),
          _full_spec(w2_root.shape),
          _full_spec(wmuls.shape),
      ],
      out_specs=[_row_spec(r, d), _row_spec(r, oc2)],
      out_shape=[
          jax.ShapeDtypeStruct((n, d), jnp.float32),
          jax.ShapeDtypeStruct((n, oc2), jnp.float32),
      ],
  )(parts, h, rdeg, b2, w2_root, wmuls)


def _tc3(parts, h2, rdeg, bmuls, wroots, r):
  n, hd = h2.shape
  oc2 = parts.shape[2]

  def body(p_ref, h2_ref, rdeg_ref, br, wr, out_ref):
    mean3 = (p_ref[0] + p_ref[1]) * rdeg_ref[...]
    out_ref[...] = (
        mean3 + br[...]
        + jnp.dot(h2_ref[...], wr[...], preferred_element_type=jnp.float32))

  return pl.pallas_call(
      body,
      grid=(n // r,),
      in_specs=[
          pl.BlockSpec((2, r, oc2), lambda i: (0, i, 0)),
          _row_spec(r, hd),
          _row_spec(r, 1),
          _full_spec(bmuls.shape),
          _full_spec(wroots.shape),
      ],
      out_specs=_row_spec(r, oc2),
      out_shape=jax.ShapeDtypeStruct((n, oc2), jnp.float32),
  )(parts, h2, rdeg, bmuls, wroots)


def kernel(x, edge_index, w1_rel, b1, w1_root, w2_rel, b2, w2_root,
           wmu_rel, bmu, wmu_root, wls_rel, bls, wls_root):
  n, din = x.shape
  e = edge_index.shape[1]
  blk = NC * NS * LANES
  ep = ((e + blk - 1) // blk) * blk
  na = ((n + 1 + NS * LANES - 1) // (NS * LANES)) * (NS * LANES)

  src = edge_index[0]
  dst = edge_index[1]
  pad = ep - e
  if pad:
    src = jnp.concatenate([src, jnp.zeros((pad,), src.dtype)])
    dst = jnp.concatenate([dst, jnp.full((pad,), n, dst.dtype)])
  src3 = src.reshape(NC * NS, -1, LANES)
  dst3 = dst.reshape(NC * NS, -1, LANES)

  r = 2000 if n % 2000 == 0 else 8 * (n // 8)
  t1 = jnp.concatenate(
      [x, jnp.ones((n, 1), x.dtype), jnp.zeros((n, 15), x.dtype)], axis=1)
  p1 = _sc_segment_sum(t1, src3, dst3, na)
  h, t, rdeg = _tc1(p1, x, w1_rel, b1.reshape(1, -1), w1_root, w2_rel, r)

  p2 = _sc_segment_sum(t, src3, dst3, na)
  wmuls = jnp.concatenate([wmu_rel, wls_rel], axis=1)
  h2, p = _tc2(p2, h, rdeg, b2.reshape(1, -1), w2_root, wmuls, r)

  p3 = _sc_segment_sum(p, src3, dst3, na)
  wroots = jnp.concatenate([wmu_root, wls_root], axis=1)
  bmuls = jnp.concatenate([bmu, bls]).reshape(1, -1)
  out = _tc3(p3, h2, rdeg, bmuls, wroots, r)
  oc = wmu_rel.shape[1]
  return out[:, :oc], out[:, oc:]
